# Initial kernel scaffold; baseline (speedup 1.0000x reference)
import jax
import jax.numpy as jnp
from jax.experimental import pallas as pl


def kernel(x, row, col, values):
    raise NotImplementedError("write your pallas kernel here")


# speedup vs baseline: 1339.6661x; 1339.6661x over previous
import functools

import jax
import jax.numpy as jnp
from jax import lax
from jax.experimental import pallas as pl
from jax.experimental.pallas import tpu as pltpu
from jax.experimental.pallas import tpu_sc as plsc

NUM_SRC = 65536
NUM_DST = 65536
BATCH = 64

NC = 2
NS = 16
CB = 16
NCHUNK = BATCH // CB
PASSES = NCHUNK // NC
BLK = 128
K = 8
EBLK = K * BLK
ZR = 1024
TROWS = NUM_DST // NS


def _splat(vec16, u):
    idx = jnp.full((16, 1), u, jnp.int32)
    dnums = lax.GatherDimensionNumbers(
        offset_dims=(), collapsed_slice_dims=(0,), start_index_map=(0,))
    return lax.gather(vec16, idx, dnums, (1,),
                      mode=lax.GatherScatterMode.PROMISE_IN_BOUNDS)


def _body(nblocks, xg, colr, rowr, valr, out,
          acc, colb, cadj, rowb, valb, rbuf, zbuf, sem_i, sem_g, sem_s):
    cid = lax.axis_index("c")
    sid = lax.axis_index("s")

    def zfill(i, _):
        zbuf[i, :] = jnp.zeros((CB,), jnp.float32)
        return 0
    lax.fori_loop(0, ZR, zfill, 0)

    for p in range(PASSES):
        chunk = cid * PASSES + p
        cbase = chunk * NUM_SRC

        for z in range(TROWS // ZR):
            pltpu.sync_copy(zbuf, acc.at[pl.ds(sid * TROWS + z * ZR, ZR)])
        plsc.subcore_barrier()

        def eblock(b, _):
            boff = (sid * nblocks + b) * K
            c_i = pltpu.async_copy(colr.at[pl.ds(boff, K)], colb, sem_i)
            c_r = pltpu.async_copy(rowr.at[pl.ds(boff, K)], rowb, sem_i)
            c_v = pltpu.async_copy(valr.at[pl.ds(boff, K)], valb, sem_i)
            c_i.wait()
            c_r.wait()
            c_v.wait()
            for j in range(K):
                for i in range(BLK // 16):
                    cadj[j, pl.ds(i * 16, 16)] = (
                        colb[j, pl.ds(i * 16, 16)] + cbase)
            gs = [pltpu.async_copy(xg.at[cadj.at[j]], rbuf.at[j], sem_g)
                  for j in range(K)]
            for g in gs:
                g.wait()
            for j in range(K):
                def mbody(t, _, j=j):
                    base = t * 16
                    vals16 = valb[j, pl.ds(base, 16)]
                    for u in range(16):
                        sp = _splat(vals16, u)
                        rbuf[j, base + u, :] = rbuf[j, base + u, :] * sp
                    return 0
                lax.fori_loop(0, BLK // 16, mbody, 0)
            ss = [pltpu.async_copy(rbuf.at[j], acc.at[rowb.at[j]], sem_s,
                                   add=True)
                  for j in range(K)]
            for s in ss:
                s.wait()
            return 0

        lax.fori_loop(0, nblocks, eblock, 0)
        plsc.subcore_barrier()

        pltpu.sync_copy(
            acc.at[pl.ds(sid * TROWS, TROWS)],
            out.at[pl.ds(chunk * NUM_DST + sid * TROWS, TROWS)])


def kernel(x, row, col, values):
    e = row.shape[0]
    eb = NS * EBLK
    e_pad = ((e + eb - 1) // eb) * eb
    pad = e_pad - e
    nblocks = e_pad // (NS * EBLK)

    colr = jnp.pad(col, (0, pad)).reshape(e_pad // BLK, BLK)
    rowr = jnp.pad(row, (0, pad)).reshape(e_pad // BLK, BLK)
    valr = jnp.pad(values, (0, pad)).reshape(e_pad // BLK, BLK)
    xg = (x.reshape(NCHUNK, CB, NUM_SRC)
          .transpose(0, 2, 1)
          .reshape(NCHUNK * NUM_SRC, CB))

    mesh = plsc.VectorSubcoreMesh(core_axis_name="c", subcore_axis_name="s")
    f = pl.kernel(
        functools.partial(_body, nblocks),
        out_type=jax.ShapeDtypeStruct((NCHUNK * NUM_DST, CB), jnp.float32),
        mesh=mesh,
        compiler_params=pltpu.CompilerParams(use_tc_tiling_on_sc=False),
        scratch_types=[
            pltpu.VMEM_SHARED((NUM_DST, CB), jnp.float32),
            pltpu.VMEM((K, BLK), jnp.int32),
            pltpu.VMEM((K, BLK), jnp.int32),
            pltpu.VMEM((K, BLK), jnp.int32),
            pltpu.VMEM((K, BLK), jnp.float32),
            pltpu.VMEM((K, BLK, CB), jnp.float32),
            pltpu.VMEM((ZR, CB), jnp.float32),
            pltpu.SemaphoreType.DMA,
            pltpu.SemaphoreType.DMA,
            pltpu.SemaphoreType.DMA,
        ],
    )
    outg = f(xg, colr, rowr, valr)
    return (outg.reshape(NCHUNK, NUM_DST, CB)
            .transpose(0, 2, 1)
            .reshape(BATCH, NUM_DST))
